# baseline (speedup 1.0000x reference)
import functools

import jax
import jax.numpy as jnp
from jax import lax
from jax.experimental import pallas as pl
from jax.experimental.pallas import tpu as pltpu
from jax.experimental.pallas import tpu_sc as plsc

N = 10000
E = 160000
D = 256
G = 64

NSUB = 16
K = 64
EPS = 10240
EPAD = NSUB * EPS
NCHUNK = EPS // K
ACC_ROWS = 10008
ZROWS = 640

BR = 2000
NB = N // BR


def _mm_split_body(x_ref, w_ref, o_ref):
    m = jnp.dot(x_ref[...], w_ref[...], preferred_element_type=jnp.float32)
    o_ref[0] = m[:, :128]
    o_ref[1] = m[:, 128:]


def _mm_split(x, w):
    return pl.pallas_call(
        _mm_split_body,
        grid=(NB,),
        in_specs=[
            pl.BlockSpec((BR, D), lambda i: (i, 0)),
            pl.BlockSpec((D, D), lambda i: (0, 0)),
        ],
        out_specs=pl.BlockSpec((2, BR, 128), lambda i: (0, i, 0)),
        out_shape=jax.ShapeDtypeStruct((2, N, 128), jnp.float32),
    )(x, w)


def _norm_mm_body(agg_ref, b_ref, w_ref, o_ref):
    a = jnp.concatenate([agg_ref[0], agg_ref[1]], axis=1)
    h = jnp.maximum(a + b_ref[...], 0.0)
    n = jnp.sqrt(jnp.sum(h * h, axis=1, keepdims=True))
    h = h / jnp.maximum(n, 1e-12)
    m = jnp.dot(h, w_ref[...], preferred_element_type=jnp.float32)
    o_ref[0] = m[:, :128]
    o_ref[1] = m[:, 128:]


def _norm_mm(agg, b, w):
    return pl.pallas_call(
        _norm_mm_body,
        grid=(NB,),
        in_specs=[
            pl.BlockSpec((2, BR, 128), lambda i: (0, i, 0)),
            pl.BlockSpec((1, D), lambda i: (0, 0)),
            pl.BlockSpec((D, D), lambda i: (0, 0)),
        ],
        out_specs=pl.BlockSpec((2, BR, 128), lambda i: (0, i, 0)),
        out_shape=jax.ShapeDtypeStruct((2, N, 128), jnp.float32),
    )(agg, b.reshape(1, D), w)


def _pool_mlp_body(agg_ref, b_ref, batch_ref, wf1_ref, bf1_ref, wf2_ref,
                   bf2_ref, wo_ref, bo_ref, o_ref, sums_ref, cnt_ref):
    i = pl.program_id(0)

    a = jnp.concatenate([agg_ref[0], agg_ref[1]], axis=1)
    h = jnp.maximum(a + b_ref[...], 0.0)
    n = jnp.sqrt(jnp.sum(h * h, axis=1, keepdims=True))
    h = h / jnp.maximum(n, 1e-12)

    bb = batch_ref[0, 0]
    oh = (bb[:, None] == lax.broadcasted_iota(jnp.int32, (BR, G), 1)
          ).astype(jnp.float32)

    @pl.when(i == 0)
    def _():
        sums_ref[...] = jnp.zeros_like(sums_ref)
        cnt_ref[...] = jnp.zeros_like(cnt_ref)

    dn = (((0,), (0,)), ((), ()))
    sums_ref[...] += lax.dot_general(oh, h, dn,
                                     preferred_element_type=jnp.float32)
    cnt_ref[...] += lax.dot_general(oh, jnp.ones((BR, 128), jnp.float32), dn,
                                    preferred_element_type=jnp.float32)

    @pl.when(i == NB - 1)
    def _():
        mean = sums_ref[...] / jnp.maximum(cnt_ref[...][:, :1], 1.0)
        f = jnp.maximum(
            jnp.dot(mean, wf1_ref[...], preferred_element_type=jnp.float32)
            + bf1_ref[...], 0.0)
        f = jnp.maximum(
            jnp.dot(f, wf2_ref[...], preferred_element_type=jnp.float32)
            + bf2_ref[...], 0.0)
        o_ref[...] = (jnp.dot(f, wo_ref[...],
                              preferred_element_type=jnp.float32)
                      + bo_ref[...])


def _pool_mlp(agg, b, batch, wf1, bf1, wf2, bf2, wo, bo):
    return pl.pallas_call(
        _pool_mlp_body,
        grid=(NB,),
        in_specs=[
            pl.BlockSpec((2, BR, 128), lambda i: (0, i, 0)),
            pl.BlockSpec((1, D), lambda i: (0, 0)),
            pl.BlockSpec((1, 1, BR), lambda i: (i, 0, 0)),
            pl.BlockSpec((D, D), lambda i: (0, 0)),
            pl.BlockSpec((1, D), lambda i: (0, 0)),
            pl.BlockSpec((D, 128), lambda i: (0, 0)),
            pl.BlockSpec((1, 128), lambda i: (0, 0)),
            pl.BlockSpec((128, 1), lambda i: (0, 0)),
            pl.BlockSpec((1, 1), lambda i: (0, 0)),
        ],
        out_specs=pl.BlockSpec((G, 1), lambda i: (0, 0)),
        out_shape=jax.ShapeDtypeStruct((G, 1), jnp.float32),
        scratch_shapes=[
            pltpu.VMEM((G, D), jnp.float32),
            pltpu.VMEM((G, 128), jnp.float32),
        ],
    )(agg, b.reshape(1, D), batch.reshape(NB, 1, BR), wf1,
      bf1.reshape(1, D), wf2, bf2.reshape(1, 128), wo.reshape(128, 1),
      bo.reshape(1, 1))


def _seg_body(mtab_hbm, sd_hbm, out_hbm, ibuf, rows4, zbuf, acc, *sems):
    c = lax.axis_index("c")
    s = lax.axis_index("s")
    isems = sems[0:8]
    gsems = sems[8:12]
    ssems = sems[12:16]

    pltpu.sync_copy(sd_hbm.at[c, s, 0], ibuf.at[0])
    for q in range(1, 5):
        pltpu.async_copy(sd_hbm.at[c, s, q], ibuf.at[q], isems[q])
    pltpu.async_copy(mtab_hbm.at[ibuf.at[0, 0]], rows4.at[0], gsems[0])
    for q in (1, 2):
        pltpu.make_async_copy(sd_hbm.at[c, s, q], ibuf.at[q],
                              isems[q]).wait()
        pltpu.async_copy(mtab_hbm.at[ibuf.at[q, 0]], rows4.at[q], gsems[q])

    for j in range(8):
        for r in range(8):
            zbuf[r, pl.ds(j * 16, 16)] = jnp.zeros((16,), jnp.float32)

    @pl.when(s < NSUB - 1)
    def _():
        @pl.loop(0, ZROWS // 8)
        def _(t):
            pltpu.sync_copy(zbuf, acc.at[pl.ds(s * ZROWS + t * 8, 8)])

    @pl.when(s == NSUB - 1)
    def _():
        @pl.loop(0, (ACC_ROWS - (NSUB - 1) * ZROWS) // 8)
        def _(t):
            pltpu.sync_copy(zbuf, acc.at[pl.ds((NSUB - 1) * ZROWS + t * 8,
                                               8)])

    plsc.subcore_barrier()

    @pl.loop(0, NCHUNK, step=8)
    def _(i):
        for b in range(8):
            j = i + b
            r = b % 4
            r3 = (b + 3) % 4
            q = b
            q3 = (b + 3) % 8
            q5 = (b + 5) % 8
            q7 = (b + 7) % 8

            pltpu.make_async_copy(mtab_hbm.at[ibuf.at[q, 0]], rows4.at[r],
                                  gsems[r]).wait()
            pltpu.async_copy(rows4.at[r], acc.at[ibuf.at[q, 1]], ssems[r],
                             add=True)

            @pl.when(j < NCHUNK - 3)
            def _():
                @pl.when(j > 0)
                def _():
                    pltpu.make_async_copy(rows4.at[r3],
                                          acc.at[ibuf.at[q7, 1]],
                                          ssems[r3]).wait()

                pltpu.make_async_copy(sd_hbm.at[c, s, j + 3], ibuf.at[q3],
                                      isems[q3]).wait()
                pltpu.async_copy(mtab_hbm.at[ibuf.at[q3, 0]], rows4.at[r3],
                                 gsems[r3])

            @pl.when(j < NCHUNK - 5)
            def _():
                pltpu.async_copy(sd_hbm.at[c, s, j + 5], ibuf.at[q5],
                                 isems[q5])

    for jj in range(NCHUNK - 4, NCHUNK):
        pltpu.make_async_copy(rows4.at[jj % 4], acc.at[ibuf.at[jj % 8, 1]],
                              ssems[jj % 4]).wait()

    plsc.subcore_barrier()

    @pl.when(s < NSUB - 1)
    def _():
        pltpu.sync_copy(acc.at[pl.ds(s * ZROWS, ZROWS)],
                        out_hbm.at[c, pl.ds(s * ZROWS, ZROWS)])

    @pl.when(s == NSUB - 1)
    def _():
        r0 = (NSUB - 1) * ZROWS
        pltpu.sync_copy(acc.at[pl.ds(r0, N - r0)],
                        out_hbm.at[c, pl.ds(r0, N - r0)])


def _segment_sum_sc(mtab, sd):
    mesh = plsc.VectorSubcoreMesh(core_axis_name="c", subcore_axis_name="s")
    k = pl.kernel(
        _seg_body,
        out_type=jax.ShapeDtypeStruct((2, N, 128), jnp.float32),
        mesh=mesh,
        scratch_types=[
            pltpu.VMEM((8, 2, K), jnp.int32),
            pltpu.VMEM((4, K, 128), jnp.float32),
            pltpu.VMEM((8, 128), jnp.float32),
            pltpu.VMEM_SHARED((ACC_ROWS, 128), jnp.float32),
        ] + [pltpu.SemaphoreType.DMA] * 16,
    )
    return k(mtab, sd)


def kernel(x, edge_index, batch, W1, b1, W2, b2, Wf1, bf1, Wf2, bf2, Wo, bo):
    src = edge_index[0].astype(jnp.int32)
    dst = edge_index[1].astype(jnp.int32)
    order = jnp.argsort(src)
    src = src[order]
    dst = dst[order]
    pad = EPAD - E
    src_p = jnp.concatenate([src, jnp.zeros((pad,), jnp.int32)])
    src2 = jnp.stack([src_p, src_p + N])
    dstp = jnp.concatenate([dst, jnp.full((pad,), N, jnp.int32)])
    s4 = src2.reshape(2, NSUB, NCHUNK, 1, K)
    d4 = jnp.broadcast_to(dstp.reshape(1, NSUB, NCHUNK, 1, K),
                          (2, NSUB, NCHUNK, 1, K))
    sd = jnp.concatenate([s4, d4], axis=3)

    m1 = _mm_split(x, W1)
    agg1 = _segment_sum_sc(m1.reshape(2 * N, 128), sd)
    m2 = _norm_mm(agg1, b1, W2)
    agg2 = _segment_sum_sc(m2.reshape(2 * N, 128), sd)
    return _pool_mlp(agg2, b2, batch.astype(jnp.int32), Wf1, bf1, Wf2, bf2,
                     Wo, bo)


# speedup vs baseline: 1.5018x; 1.5018x over previous
import functools

import jax
import jax.numpy as jnp
from jax import lax
from jax.experimental import pallas as pl
from jax.experimental.pallas import tpu as pltpu
from jax.experimental.pallas import tpu_sc as plsc

N = 10000
E = 160000
D = 256
G = 64

NSUB = 16
K = 80
EPS = 10240
EPAD = NSUB * EPS
NCHUNK = EPS // K
ACC_ROWS = 10008
ZROWS = 640

BR = 2000
NB = N // BR


def _mm_split_body(x_ref, w_ref, o_ref):
    m = jnp.dot(x_ref[...], w_ref[...], preferred_element_type=jnp.float32)
    o_ref[0] = m[:, :128]
    o_ref[1] = m[:, 128:]


def _mm_split(x, w):
    return pl.pallas_call(
        _mm_split_body,
        grid=(NB,),
        in_specs=[
            pl.BlockSpec((BR, D), lambda i: (i, 0)),
            pl.BlockSpec((D, D), lambda i: (0, 0)),
        ],
        out_specs=pl.BlockSpec((2, BR, 128), lambda i: (0, i, 0)),
        out_shape=jax.ShapeDtypeStruct((2, N, 128), jnp.float32),
    )(x, w)


def _norm_mm_body(agg_ref, b_ref, w_ref, o_ref):
    a = jnp.concatenate([agg_ref[0], agg_ref[1]], axis=1)
    h = jnp.maximum(a + b_ref[...], 0.0)
    n = jnp.sqrt(jnp.sum(h * h, axis=1, keepdims=True))
    h = h / jnp.maximum(n, 1e-12)
    m = jnp.dot(h, w_ref[...], preferred_element_type=jnp.float32)
    o_ref[0] = m[:, :128]
    o_ref[1] = m[:, 128:]


def _norm_mm(agg, b, w):
    return pl.pallas_call(
        _norm_mm_body,
        grid=(NB,),
        in_specs=[
            pl.BlockSpec((2, BR, 128), lambda i: (0, i, 0)),
            pl.BlockSpec((1, D), lambda i: (0, 0)),
            pl.BlockSpec((D, D), lambda i: (0, 0)),
        ],
        out_specs=pl.BlockSpec((2, BR, 128), lambda i: (0, i, 0)),
        out_shape=jax.ShapeDtypeStruct((2, N, 128), jnp.float32),
    )(agg, b.reshape(1, D), w)


def _pool_mlp_body(agg_ref, b_ref, batch_ref, wf1_ref, bf1_ref, wf2_ref,
                   bf2_ref, wo_ref, bo_ref, o_ref, sums_ref, cnt_ref):
    i = pl.program_id(0)

    a = jnp.concatenate([agg_ref[0], agg_ref[1]], axis=1)
    h = jnp.maximum(a + b_ref[...], 0.0)
    n = jnp.sqrt(jnp.sum(h * h, axis=1, keepdims=True))
    h = h / jnp.maximum(n, 1e-12)

    bb = batch_ref[0, 0]
    oh = (bb[:, None] == lax.broadcasted_iota(jnp.int32, (BR, G), 1)
          ).astype(jnp.float32)

    @pl.when(i == 0)
    def _():
        sums_ref[...] = jnp.zeros_like(sums_ref)
        cnt_ref[...] = jnp.zeros_like(cnt_ref)

    dn = (((0,), (0,)), ((), ()))
    sums_ref[...] += lax.dot_general(oh, h, dn,
                                     preferred_element_type=jnp.float32)
    cnt_ref[...] += lax.dot_general(oh, jnp.ones((BR, 128), jnp.float32), dn,
                                    preferred_element_type=jnp.float32)

    @pl.when(i == NB - 1)
    def _():
        mean = sums_ref[...] / jnp.maximum(cnt_ref[...][:, :1], 1.0)
        f = jnp.maximum(
            jnp.dot(mean, wf1_ref[...], preferred_element_type=jnp.float32)
            + bf1_ref[...], 0.0)
        f = jnp.maximum(
            jnp.dot(f, wf2_ref[...], preferred_element_type=jnp.float32)
            + bf2_ref[...], 0.0)
        o_ref[...] = (jnp.dot(f, wo_ref[...],
                              preferred_element_type=jnp.float32)
                      + bo_ref[...])


def _pool_mlp(agg, b, batch, wf1, bf1, wf2, bf2, wo, bo):
    return pl.pallas_call(
        _pool_mlp_body,
        grid=(NB,),
        in_specs=[
            pl.BlockSpec((2, BR, 128), lambda i: (0, i, 0)),
            pl.BlockSpec((1, D), lambda i: (0, 0)),
            pl.BlockSpec((1, 1, BR), lambda i: (i, 0, 0)),
            pl.BlockSpec((D, D), lambda i: (0, 0)),
            pl.BlockSpec((1, D), lambda i: (0, 0)),
            pl.BlockSpec((D, 128), lambda i: (0, 0)),
            pl.BlockSpec((1, 128), lambda i: (0, 0)),
            pl.BlockSpec((128, 1), lambda i: (0, 0)),
            pl.BlockSpec((1, 1), lambda i: (0, 0)),
        ],
        out_specs=pl.BlockSpec((G, 1), lambda i: (0, 0)),
        out_shape=jax.ShapeDtypeStruct((G, 1), jnp.float32),
        scratch_shapes=[
            pltpu.VMEM((G, D), jnp.float32),
            pltpu.VMEM((G, 128), jnp.float32),
        ],
    )(agg, b.reshape(1, D), batch.reshape(NB, 1, BR), wf1,
      bf1.reshape(1, D), wf2, bf2.reshape(1, 128), wo.reshape(128, 1),
      bo.reshape(1, 1))


def _seg_body(mtab_hbm, sd_hbm, out_hbm, ibuf, rows4, zbuf, acc, *sems):
    c = lax.axis_index("c")
    s = lax.axis_index("s")
    isems = sems[0:8]
    gsems = sems[8:12]
    ssems = sems[12:16]

    pltpu.sync_copy(sd_hbm.at[c, s, 0], ibuf.at[0])
    for q in range(1, 5):
        pltpu.async_copy(sd_hbm.at[c, s, q], ibuf.at[q], isems[q])
    pltpu.async_copy(mtab_hbm.at[ibuf.at[0, 0]], rows4.at[0], gsems[0])
    for q in (1, 2):
        pltpu.make_async_copy(sd_hbm.at[c, s, q], ibuf.at[q],
                              isems[q]).wait()
        pltpu.async_copy(mtab_hbm.at[ibuf.at[q, 0]], rows4.at[q], gsems[q])

    for j in range(8):
        for r in range(8):
            zbuf[r, pl.ds(j * 16, 16)] = jnp.zeros((16,), jnp.float32)

    @pl.when(s < NSUB - 1)
    def _():
        @pl.loop(0, ZROWS // 8)
        def _(t):
            pltpu.sync_copy(zbuf, acc.at[pl.ds(s * ZROWS + t * 8, 8)])

    @pl.when(s == NSUB - 1)
    def _():
        @pl.loop(0, (ACC_ROWS - (NSUB - 1) * ZROWS) // 8)
        def _(t):
            pltpu.sync_copy(zbuf, acc.at[pl.ds((NSUB - 1) * ZROWS + t * 8,
                                               8)])

    plsc.subcore_barrier()

    @pl.loop(0, NCHUNK, step=8)
    def _(i):
        for b in range(8):
            j = i + b
            r = b % 4
            r3 = (b + 3) % 4
            q = b
            q3 = (b + 3) % 8
            q5 = (b + 5) % 8
            q7 = (b + 7) % 8

            pltpu.make_async_copy(mtab_hbm.at[ibuf.at[q, 0]], rows4.at[r],
                                  gsems[r]).wait()
            pltpu.async_copy(rows4.at[r], acc.at[ibuf.at[q, 1]], ssems[r],
                             add=True)

            @pl.when(j < NCHUNK - 3)
            def _():
                @pl.when(j > 0)
                def _():
                    pltpu.make_async_copy(rows4.at[r3],
                                          acc.at[ibuf.at[q7, 1]],
                                          ssems[r3]).wait()

                pltpu.make_async_copy(sd_hbm.at[c, s, j + 3], ibuf.at[q3],
                                      isems[q3]).wait()
                pltpu.async_copy(mtab_hbm.at[ibuf.at[q3, 0]], rows4.at[r3],
                                 gsems[r3])

            @pl.when(j < NCHUNK - 5)
            def _():
                pltpu.async_copy(sd_hbm.at[c, s, j + 5], ibuf.at[q5],
                                 isems[q5])

    for jj in range(NCHUNK - 4, NCHUNK):
        pltpu.make_async_copy(rows4.at[jj % 4], acc.at[ibuf.at[jj % 8, 1]],
                              ssems[jj % 4]).wait()

    plsc.subcore_barrier()

    @pl.when(s < NSUB - 1)
    def _():
        pltpu.sync_copy(acc.at[pl.ds(s * ZROWS, ZROWS)],
                        out_hbm.at[c, pl.ds(s * ZROWS, ZROWS)])

    @pl.when(s == NSUB - 1)
    def _():
        r0 = (NSUB - 1) * ZROWS
        pltpu.sync_copy(acc.at[pl.ds(r0, N - r0)],
                        out_hbm.at[c, pl.ds(r0, N - r0)])


def _segment_sum_sc(mtab, sd):
    mesh = plsc.VectorSubcoreMesh(core_axis_name="c", subcore_axis_name="s")
    k = pl.kernel(
        _seg_body,
        out_type=jax.ShapeDtypeStruct((2, N, 128), jnp.float32),
        mesh=mesh,
        scratch_types=[
            pltpu.VMEM((8, 2, K), jnp.int32),
            pltpu.VMEM((4, K, 128), jnp.float32),
            pltpu.VMEM((8, 128), jnp.float32),
            pltpu.VMEM_SHARED((ACC_ROWS, 128), jnp.float32),
        ] + [pltpu.SemaphoreType.DMA] * 16,
    )
    return k(mtab, sd)


def kernel(x, edge_index, batch, W1, b1, W2, b2, Wf1, bf1, Wf2, bf2, Wo, bo):
    src = edge_index[0].astype(jnp.int32)
    dst = edge_index[1].astype(jnp.int32)
    pad = EPAD - E
    src_p = jnp.concatenate([src, jnp.zeros((pad,), jnp.int32)])
    src2 = jnp.stack([src_p, src_p + N])
    dstp = jnp.concatenate([dst, jnp.full((pad,), N, jnp.int32)])
    s4 = src2.reshape(2, NSUB, NCHUNK, 1, K)
    d4 = jnp.broadcast_to(dstp.reshape(1, NSUB, NCHUNK, 1, K),
                          (2, NSUB, NCHUNK, 1, K))
    sd = jnp.concatenate([s4, d4], axis=3)

    m1 = _mm_split(x, W1)
    agg1 = _segment_sum_sc(m1.reshape(2 * N, 128), sd)
    m2 = _norm_mm(agg1, b1, W2)
    agg2 = _segment_sum_sc(m2.reshape(2 * N, 128), sd)
    return _pool_mlp(agg2, b2, batch.astype(jnp.int32), Wf1, bf1, Wf2, bf2,
                     Wo, bo)
